# Initial kernel scaffold; baseline (speedup 1.0000x reference)
import jax
import jax.numpy as jnp
from jax.experimental import pallas as pl


def kernel(hidden_states, selected_experts, routing_weights, W_gate, W_up, W_down):
    raise NotImplementedError("write your pallas kernel here")


# speedup vs baseline: 1.1472x; 1.1472x over previous
import functools

import jax
import jax.numpy as jnp
from jax import lax
from jax.experimental import pallas as pl
from jax.experimental.pallas import tpu as pltpu
from jax.experimental.pallas import tpu_sc as plsc

E = 64
K = 2
H = 1024
F = 512
BLK = 64
CAPB = 4
P = 8192
NW = 32

def _sc_mesh():
    return plsc.VectorSubcoreMesh(core_axis_name="c", subcore_axis_name="s")


def _sc_permute(hidden, src, dst):
    R = src.shape[0]
    per_w = R // NW
    CH = 64

    @functools.partial(
        pl.kernel,
        out_type=jax.ShapeDtypeStruct((P, H), jnp.float32),
        mesh=_sc_mesh(),
        scratch_types=[
            pltpu.VMEM((CH,), jnp.int32),
            pltpu.VMEM((CH,), jnp.int32),
            pltpu.VMEM((CH, H), jnp.float32),
            pltpu.SemaphoreType.DMA,
        ],
    )
    def body(hid_hbm, src_hbm, dst_hbm, xp_hbm, src_v, dst_v, rows_v, sem):
        wid = lax.axis_index("s") * 2 + lax.axis_index("c")
        base = wid * per_w
        for c in range(per_w // CH):
            o = base + c * CH
            pltpu.sync_copy(src_hbm.at[pl.ds(o, CH)], src_v)
            pltpu.async_copy(hid_hbm.at[src_v], rows_v, sem).wait()
            pltpu.sync_copy(dst_hbm.at[pl.ds(o, CH)], dst_v)
            pltpu.async_copy(rows_v, xp_hbm.at[dst_v], sem).wait()

    return body(hidden, src, dst)


def _sc_combine(y_p, p0, p1):
    T = p0.shape[0]
    per_w = T // NW
    CT = 32

    @functools.partial(
        pl.kernel,
        out_type=jax.ShapeDtypeStruct((T, H), jnp.float32),
        mesh=_sc_mesh(),
        scratch_types=[
            pltpu.VMEM((CT,), jnp.int32),
            pltpu.VMEM((CT,), jnp.int32),
            pltpu.VMEM((CT, H), jnp.float32),
            pltpu.VMEM((CT, H), jnp.float32),
            pltpu.SemaphoreType.DMA,
        ],
    )
    def body(yp_hbm, p0_hbm, p1_hbm, out_hbm, i0_v, i1_v, a_v, b_v, sem):
        wid = lax.axis_index("s") * 2 + lax.axis_index("c")
        tbase = wid * per_w
        for c in range(per_w // CT):
            o = tbase + c * CT
            pltpu.sync_copy(p0_hbm.at[pl.ds(o, CT)], i0_v)
            pltpu.async_copy(yp_hbm.at[i0_v], a_v, sem).wait()
            pltpu.sync_copy(p1_hbm.at[pl.ds(o, CT)], i1_v)
            pltpu.async_copy(yp_hbm.at[i1_v], b_v, sem).wait()

            def row_add(r, carry):
                for s in range(H // 16):
                    sl = pl.ds(s * 16, 16)
                    a_v[r, sl] = a_v[r, sl] + b_v[r, sl]
                return carry

            lax.fori_loop(0, CT, row_add, 0)
            pltpu.sync_copy(a_v, out_hbm.at[pl.ds(o, CT)])

    return body(y_p, p0, p1)


def _tc_body(bidx_ref, nblk_ref, x_ref, wg_ref, wu_ref, wd_ref, wrow_ref,
             y_ref):
    e = pl.program_id(0)
    j = pl.program_id(1)

    @pl.when(j < nblk_ref[e])
    def _():
        x = x_ref[...]
        g = jnp.dot(x, wg_ref[0], preferred_element_type=jnp.float32)
        u = jnp.dot(x, wu_ref[0], preferred_element_type=jnp.float32)
        h = (g * jax.nn.sigmoid(g)) * u * wrow_ref[...]
        y_ref[...] = jnp.dot(h, wd_ref[0], preferred_element_type=jnp.float32)


def _tc_moe(bidx, nblk, x_p, W_gate, W_up, W_down, w_p, interpret=False):
    grid_spec = pltpu.PrefetchScalarGridSpec(
        num_scalar_prefetch=2,
        grid=(E, CAPB),
        in_specs=[
            pl.BlockSpec((BLK, H), lambda e, j, bidx, nblk: (bidx[e * CAPB + j], 0)),
            pl.BlockSpec((1, H, F), lambda e, j, bidx, nblk: (e, 0, 0)),
            pl.BlockSpec((1, H, F), lambda e, j, bidx, nblk: (e, 0, 0)),
            pl.BlockSpec((1, F, H), lambda e, j, bidx, nblk: (e, 0, 0)),
            pl.BlockSpec((BLK, 1), lambda e, j, bidx, nblk: (bidx[e * CAPB + j], 0)),
        ],
        out_specs=pl.BlockSpec((BLK, H), lambda e, j, bidx, nblk: (bidx[e * CAPB + j], 0)),
    )
    return pl.pallas_call(
        _tc_body,
        grid_spec=grid_spec,
        out_shape=jax.ShapeDtypeStruct((P, H), jnp.float32),
        interpret=interpret,
    )(bidx, nblk, x_p, W_gate, W_up, W_down, w_p)


def _routing_meta(selected_experts, routing_weights):
    R = selected_experts.shape[0] * K
    eid = selected_experts.reshape(-1).astype(jnp.int32)
    order = jnp.argsort(eid).astype(jnp.int32)
    seg = eid[order]
    sizes = jnp.bincount(eid, length=E).astype(jnp.int32)
    nblk = jnp.minimum((sizes + BLK - 1) // BLK, CAPB).astype(jnp.int32)
    psize = nblk * BLK
    poff = (jnp.cumsum(psize) - psize).astype(jnp.int32)
    off = (jnp.cumsum(sizes) - sizes).astype(jnp.int32)
    rank = jnp.arange(R, dtype=jnp.int32) - off[seg]
    valid = rank < psize[seg]
    dump = (P - BLK) + (jnp.arange(R, dtype=jnp.int32) % BLK)
    dst = jnp.where(valid, poff[seg] + rank, dump).astype(jnp.int32)
    src = (order // K).astype(jnp.int32)

    w_sorted = routing_weights.reshape(-1)[order]
    w_p = jnp.zeros((P,), jnp.float32).at[dst].set(w_sorted).reshape(P, 1)

    inv = jnp.zeros((R,), jnp.int32).at[order].set(dst)
    p0 = inv[0::K]
    p1 = inv[1::K]

    act = (jnp.arange(CAPB, dtype=jnp.int32)[None, :] < nblk[:, None]).reshape(-1)
    bi = (poff[:, None] // BLK
          + jnp.arange(CAPB, dtype=jnp.int32)[None, :]).reshape(-1)
    pos = jnp.where(act, jnp.arange(E * CAPB, dtype=jnp.int32), -1)
    last = lax.associative_scan(jnp.maximum, pos)
    first_bi = bi[jnp.argmax(act)]
    bidx = jnp.where(last >= 0, bi[jnp.maximum(last, 0)], first_bi).astype(jnp.int32)
    return src, dst, w_p, p0, p1, bidx, nblk


def kernel(hidden_states, selected_experts, routing_weights,
           W_gate, W_up, W_down):
    src, dst, w_p, p0, p1, bidx, nblk = _routing_meta(
        selected_experts, routing_weights)
    x_p = _sc_permute(hidden_states, src, dst)
    y_p = _tc_moe(bidx, nblk, x_p, W_gate, W_up, W_down, w_p)
    return _sc_combine(y_p, p0, p1)


# speedup vs baseline: 1.1480x; 1.0007x over previous
import functools

import jax
import jax.numpy as jnp
from jax import lax
from jax.experimental import pallas as pl
from jax.experimental.pallas import tpu as pltpu
from jax.experimental.pallas import tpu_sc as plsc

E = 64
K = 2
H = 1024
F = 512
BLK = 64
CAPB = 4
P = 8192
NW = 32

def _sc_mesh():
    return plsc.VectorSubcoreMesh(core_axis_name="c", subcore_axis_name="s")


def _sc_permute(hidden, src, dst):
    R = src.shape[0]
    per_w = R // NW
    CH = 64

    @functools.partial(
        pl.kernel,
        out_type=jax.ShapeDtypeStruct((P, H), jnp.float32),
        mesh=_sc_mesh(),
        scratch_types=[
            pltpu.VMEM((CH,), jnp.int32),
            pltpu.VMEM((CH,), jnp.int32),
            pltpu.VMEM((CH, H), jnp.float32),
            pltpu.SemaphoreType.DMA,
        ],
    )
    def body(hid_hbm, src_hbm, dst_hbm, xp_hbm, src_v, dst_v, rows_v, sem):
        wid = lax.axis_index("s") * 2 + lax.axis_index("c")
        base = wid * per_w
        for c in range(per_w // CH):
            o = base + c * CH
            pltpu.sync_copy(src_hbm.at[pl.ds(o, CH)], src_v)
            pltpu.async_copy(hid_hbm.at[src_v], rows_v, sem).wait()
            pltpu.sync_copy(dst_hbm.at[pl.ds(o, CH)], dst_v)
            pltpu.async_copy(rows_v, xp_hbm.at[dst_v], sem).wait()

    return body(hidden, src, dst)


def _sc_combine(y_p, p0, p1):
    T = p0.shape[0]
    per_w = T // NW
    CT = 32

    @functools.partial(
        pl.kernel,
        out_type=jax.ShapeDtypeStruct((T, H), jnp.float32),
        mesh=_sc_mesh(),
        scratch_types=[
            pltpu.VMEM((CT,), jnp.int32),
            pltpu.VMEM((CT,), jnp.int32),
            pltpu.VMEM((CT, H), jnp.float32),
            pltpu.VMEM((CT, H), jnp.float32),
            pltpu.SemaphoreType.DMA,
        ],
    )
    def body(yp_hbm, p0_hbm, p1_hbm, out_hbm, i0_v, i1_v, a_v, b_v, sem):
        wid = lax.axis_index("s") * 2 + lax.axis_index("c")
        tbase = wid * per_w
        for c in range(per_w // CT):
            o = tbase + c * CT
            pltpu.sync_copy(p0_hbm.at[pl.ds(o, CT)], i0_v)
            pltpu.async_copy(yp_hbm.at[i0_v], a_v, sem).wait()
            pltpu.sync_copy(p1_hbm.at[pl.ds(o, CT)], i1_v)
            pltpu.async_copy(yp_hbm.at[i1_v], b_v, sem).wait()

            def row_add(r, carry):
                for s in range(H // 16):
                    sl = pl.ds(s * 16, 16)
                    a_v[r, sl] = a_v[r, sl] + b_v[r, sl]
                return carry

            lax.fori_loop(0, CT, row_add, 0)
            pltpu.sync_copy(a_v, out_hbm.at[pl.ds(o, CT)])

    return body(y_p, p0, p1)


def _tc_body(bidx_ref, nblk_ref, x_ref, wg_ref, wu_ref, wd_ref, wrow_ref,
             y_ref):
    e = pl.program_id(0)
    j = pl.program_id(1)

    @pl.when(j < nblk_ref[e])
    def _():
        x = x_ref[...]
        g = jnp.dot(x, wg_ref[0], preferred_element_type=jnp.float32)
        u = jnp.dot(x, wu_ref[0], preferred_element_type=jnp.float32)
        h = (g * jax.nn.sigmoid(g)) * u * wrow_ref[...]
        y_ref[...] = jnp.dot(h, wd_ref[0], preferred_element_type=jnp.float32)


def _tc_moe(bidx, nblk, x_p, W_gate, W_up, W_down, w_p):
    grid_spec = pltpu.PrefetchScalarGridSpec(
        num_scalar_prefetch=2,
        grid=(E, CAPB),
        in_specs=[
            pl.BlockSpec((BLK, H), lambda e, j, bidx, nblk: (bidx[e * CAPB + j], 0)),
            pl.BlockSpec((1, H, F), lambda e, j, bidx, nblk: (e, 0, 0)),
            pl.BlockSpec((1, H, F), lambda e, j, bidx, nblk: (e, 0, 0)),
            pl.BlockSpec((1, F, H), lambda e, j, bidx, nblk: (e, 0, 0)),
            pl.BlockSpec((BLK, 1), lambda e, j, bidx, nblk: (bidx[e * CAPB + j], 0)),
        ],
        out_specs=pl.BlockSpec((BLK, H), lambda e, j, bidx, nblk: (bidx[e * CAPB + j], 0)),
    )
    return pl.pallas_call(
        _tc_body,
        grid_spec=grid_spec,
        out_shape=jax.ShapeDtypeStruct((P, H), jnp.float32),
    )(bidx, nblk, x_p, W_gate, W_up, W_down, w_p)


def _routing_meta(selected_experts, routing_weights):
    R = selected_experts.shape[0] * K
    eid = selected_experts.reshape(-1).astype(jnp.int32)
    order = jnp.argsort(eid).astype(jnp.int32)
    seg = eid[order]
    sizes = jnp.bincount(eid, length=E).astype(jnp.int32)
    nblk = jnp.minimum((sizes + BLK - 1) // BLK, CAPB).astype(jnp.int32)
    psize = nblk * BLK
    poff = (jnp.cumsum(psize) - psize).astype(jnp.int32)
    off = (jnp.cumsum(sizes) - sizes).astype(jnp.int32)
    rank = jnp.arange(R, dtype=jnp.int32) - off[seg]
    valid = rank < psize[seg]
    dump = (P - BLK) + (jnp.arange(R, dtype=jnp.int32) % BLK)
    dst = jnp.where(valid, poff[seg] + rank, dump).astype(jnp.int32)
    src = (order // K).astype(jnp.int32)

    w_sorted = routing_weights.reshape(-1)[order]
    w_p = jnp.zeros((P,), jnp.float32).at[dst].set(w_sorted).reshape(P, 1)

    inv = jnp.zeros((R,), jnp.int32).at[order].set(dst)
    p0 = inv[0::K]
    p1 = inv[1::K]

    act = (jnp.arange(CAPB, dtype=jnp.int32)[None, :] < nblk[:, None]).reshape(-1)
    bi = (poff[:, None] // BLK
          + jnp.arange(CAPB, dtype=jnp.int32)[None, :]).reshape(-1)
    pos = jnp.where(act, jnp.arange(E * CAPB, dtype=jnp.int32), -1)
    last = lax.associative_scan(jnp.maximum, pos)
    first_bi = bi[jnp.argmax(act)]
    bidx = jnp.where(last >= 0, bi[jnp.maximum(last, 0)], first_bi).astype(jnp.int32)
    return src, dst, w_p, p0, p1, bidx, nblk


def kernel(hidden_states, selected_experts, routing_weights,
           W_gate, W_up, W_down):
    src, dst, w_p, p0, p1, bidx, nblk = _routing_meta(
        selected_experts, routing_weights)
    x_p = _sc_permute(hidden_states, src, dst)
    y_p = _tc_moe(bidx, nblk, x_p, W_gate, W_up, W_down, w_p)
    return _sc_combine(y_p, p0, p1)
